# baseline (speedup 1.0000x reference)
import functools

import jax
import jax.numpy as jnp
from jax import lax
from jax.experimental import pallas as pl
from jax.experimental.pallas import tpu as pltpu

B, S, D, E, K, H = 1, 2048, 768, 8, 2, 12
F = 4 * D
DH = D // H

SBLK = 512
NS = S // SBLK


def _pool_dense_kernel(x_ref, rw_ref, rb_ref, w1_ref, b1_ref, w2_ref, b2_ref,
                       out_ref):
    e = pl.program_id(0)
    s = pl.program_id(1)
    xb = x_ref[...]
    logits = jnp.dot(xb, rw_ref[...], preferred_element_type=jnp.float32)
    logits = logits + rb_ref[...]
    ecol = lax.broadcasted_iota(jnp.int32, (SBLK, E), 1)
    cnt = jnp.zeros((SBLK, E), jnp.float32)
    for j in range(E):
        lj = logits[:, j:j + 1]
        beats = (lj > logits) | ((lj == logits) & (j < ecol))
        cnt = cnt + beats.astype(jnp.float32)
    mask_all = cnt < float(K)
    sel = jnp.where(mask_all & (ecol == e), 1.0, 0.0)
    mask_e = jnp.sum(sel, axis=1, keepdims=True)

    h = jnp.dot(xb.astype(jnp.bfloat16), w1_ref[0],
                preferred_element_type=jnp.float32)
    h = h + b1_ref[0]
    h = 0.5 * h * (1.0 + lax.erf(h * (2.0 ** -0.5)))
    y = jnp.dot(h.astype(jnp.bfloat16), w2_ref[0],
                preferred_element_type=jnp.float32)
    y = y + b2_ref[0]
    contrib = (mask_e * (1.0 / K)) * y

    sl = pl.ds(s * SBLK, SBLK)

    @pl.when(e == 0)
    def _():
        out_ref[sl, :] = contrib

    @pl.when(e > 0)
    def _():
        out_ref[sl, :] = out_ref[sl, :] + contrib


def _qkv_kernel(no_ref, w_ref, b_ref, out_ref):
    out_ref[...] = lax.dot_general(
        no_ref[...].astype(jnp.bfloat16), w_ref[...].astype(jnp.bfloat16),
        (((1,), (1,)), ((), ())),
        preferred_element_type=jnp.float32) + b_ref[...]


def _attn_kernel(q_ref, k_ref, v_ref, out_ref):
    for i in range(2):
        cols = slice(i * DH, (i + 1) * DH)
        q = q_ref[:, cols].astype(jnp.bfloat16)
        k = k_ref[:, cols].astype(jnp.bfloat16)
        v = v_ref[:, cols].astype(jnp.bfloat16)
        s = lax.dot_general(q, k, (((1,), (1,)), ((), ())),
                            preferred_element_type=jnp.float32)
        s = s * (1.0 / (DH ** 0.5))
        m = jnp.max(s, axis=1, keepdims=True)
        p = jnp.exp(s - m)
        p = p / jnp.sum(p, axis=1, keepdims=True)
        out_ref[:, cols] = jnp.dot(p.astype(jnp.bfloat16), v,
                                   preferred_element_type=jnp.float32)


def _layernorm(x, w, b, eps=1e-5):
    mu = jnp.mean(x, axis=-1, keepdims=True)
    xc = x - mu
    var = jnp.mean(xc * xc, axis=-1, keepdims=True)
    return xc * jax.lax.rsqrt(var + eps) * w + b


def _final_kernel(x_ref, no_ref, ao_ref, w_ref, b_ref, l1w_ref, l1b_ref,
                  l2w_ref, l2b_ref, out_ref):
    attn_out = lax.dot_general(
        ao_ref[...].astype(jnp.bfloat16), w_ref[...].astype(jnp.bfloat16),
        (((1,), (1,)), ((), ())),
        preferred_element_type=jnp.float32) + b_ref[...]
    x1 = _layernorm(x_ref[...] + attn_out, l1w_ref[...], l1b_ref[...])
    out_ref[...] = _layernorm(x1 + no_ref[...], l2w_ref[...], l2b_ref[...])


def kernel(x, router_w, router_b, W1, b1, W2, b2, in_proj_w, in_proj_b,
           out_proj_w, out_proj_b, ln1_w, ln1_b, ln2_w, ln2_b):
    x2 = x.reshape(S, D)
    rb2 = router_b.reshape(1, E)

    neuron_outputs = pl.pallas_call(
        _pool_dense_kernel,
        grid=(E, NS),
        in_specs=[
            pl.BlockSpec((SBLK, D), lambda e, s: (s, 0)),
            pl.BlockSpec((D, E), lambda e, s: (0, 0)),
            pl.BlockSpec((1, E), lambda e, s: (0, 0)),
            pl.BlockSpec((1, D, F), lambda e, s: (e, 0, 0)),
            pl.BlockSpec((1, 1, F), lambda e, s: (e, 0, 0)),
            pl.BlockSpec((1, F, D), lambda e, s: (e, 0, 0)),
            pl.BlockSpec((1, 1, D), lambda e, s: (e, 0, 0)),
        ],
        out_specs=pl.BlockSpec((S, D), lambda e, s: (0, 0)),
        out_shape=jax.ShapeDtypeStruct((S, D), jnp.float32),
    )(x2, router_w, rb2, W1.astype(jnp.bfloat16), b1.reshape(E, 1, F),
      W2.astype(jnp.bfloat16), b2.reshape(E, 1, D))

    qkv = pl.pallas_call(
        _qkv_kernel,
        out_shape=jax.ShapeDtypeStruct((S, 3 * D), jnp.float32),
    )(neuron_outputs, in_proj_w, in_proj_b.reshape(1, 3 * D))

    ao = pl.pallas_call(
        _attn_kernel,
        grid=(H // 2,),
        in_specs=[
            pl.BlockSpec((S, 2 * DH), lambda h: (0, h)),
            pl.BlockSpec((S, 2 * DH), lambda h: (0, H // 2 + h)),
            pl.BlockSpec((S, 2 * DH), lambda h: (0, H + h)),
        ],
        out_specs=pl.BlockSpec((S, 2 * DH), lambda h: (0, h)),
        out_shape=jax.ShapeDtypeStruct((S, D), jnp.float32),
    )(qkv, qkv, qkv)

    out = pl.pallas_call(
        _final_kernel,
        out_shape=jax.ShapeDtypeStruct((S, D), jnp.float32),
    )(x2, neuron_outputs, ao, out_proj_w, out_proj_b.reshape(1, D),
      ln1_w.reshape(1, D), ln1_b.reshape(1, D),
      ln2_w.reshape(1, D), ln2_b.reshape(1, D))

    return out.reshape(B, S, D)


# speedup vs baseline: 1.0172x; 1.0172x over previous
import functools

import jax
import jax.numpy as jnp
from jax import lax
from jax.experimental import pallas as pl
from jax.experimental.pallas import tpu as pltpu
from jax.experimental.pallas import tpu_sc as plsc

B, S, D, E, K, H = 1, 2048, 768, 8, 2, 12
F = 4 * D
DH = D // H

BLK = 256
NBLK = (S * K + E * (BLK - 1) + BLK - 1) // BLK
P = NBLK * BLK

NC, NS = 2, 16
NW = NC * NS
RPW = P // NW
GCH = 64
TPW = S // NW
CCH = 32

CB = 512


def _dispatch_kernel(x_ref, rw_ref, rb_ref, dest_ref, bexp_ref, nblk_ref):
    xb = x_ref[...]
    logits = jnp.dot(xb, rw_ref[...], preferred_element_type=jnp.float32)
    logits = logits + rb_ref[...]
    ecol = lax.broadcasted_iota(jnp.int32, (S, E), 1)
    cnt = jnp.zeros((S, E), jnp.float32)
    for j in range(E):
        lj = logits[:, j:j + 1]
        beats = (lj > logits) | ((lj == logits) & (j < ecol))
        cnt = cnt + beats.astype(jnp.float32)
    mask = jnp.where(cnt < float(K), 1.0, 0.0)

    r_i = lax.broadcasted_iota(jnp.int32, (CB, CB), 0)
    c_i = lax.broadcasted_iota(jnp.int32, (CB, CB), 1)
    tri = jnp.where(r_i >= c_i, 1.0, 0.0)
    carry = jnp.zeros((1, E), jnp.float32)
    chunks = []
    for c in range(S // CB):
        mc = mask[c * CB:(c + 1) * CB, :]
        posc = jnp.dot(tri, mc, preferred_element_type=jnp.float32) + carry
        carry = posc[CB - 1:CB, :]
        chunks.append(posc)
    pos = jnp.concatenate(chunks, axis=0)
    tot = pos[S - 1:S, :]

    pc = jnp.floor((tot + float(BLK - 1)) * (1.0 / BLK)) * float(BLK)
    r8 = lax.broadcasted_iota(jnp.int32, (E, E), 0)
    c8 = lax.broadcasted_iota(jnp.int32, (E, E), 1)
    ut = jnp.where(r8 < c8, 1.0, 0.0)
    off = jnp.dot(pc, ut, preferred_element_type=jnp.float32)

    slot = off + pos - 1.0
    d0 = jnp.sum(jnp.where(cnt == 0.0, slot, 0.0), axis=1, keepdims=True)
    d1 = jnp.sum(jnp.where(cnt == 1.0, slot, 0.0), axis=1, keepdims=True)
    dest_ref[...] = jnp.concatenate([d0, d1], axis=1).astype(jnp.int32)

    bB = lax.broadcasted_iota(jnp.int32, (NBLK, E), 0).astype(jnp.float32)
    bB = bB * float(BLK)
    ind = jnp.where((bB >= off) & (bB < off + pc), 1.0, 0.0)
    ecolf = lax.broadcasted_iota(jnp.int32, (NBLK, E), 1).astype(jnp.float32)
    val = jnp.sum(ind * ecolf, axis=1, keepdims=True)
    has = jnp.sum(ind, axis=1, keepdims=True)
    bexp_ref[...] = (val + float(E - 1) * (1.0 - has)).astype(jnp.int32)
    nblk_ref[...] = (jnp.sum(pc, axis=1, keepdims=True) *
                     (1.0 / BLK)).astype(jnp.int32)


def _gather_body(x_hbm, dflat_hbm, xg_hbm, dflat_v, tok_v, buf_v, sem):
    wid = lax.axis_index("s") * NC + lax.axis_index("c")
    base = wid * RPW
    pltpu.sync_copy(dflat_hbm, dflat_v)
    for cc in range(RPW // 16):
        tok_v[pl.ds(cc * 16, 16)] = jnp.zeros((16,), jnp.int32)

    def body(i, carry):
        d = dflat_v[pl.ds(i * 16, 16)]
        a = lax.iota(jnp.int32, 16) + i * 16
        t = lax.bitwise_and(a, S - 1)
        drel = d - base
        m = (drel >= 0) & (drel < RPW)
        drel_c = jnp.clip(drel, 0, RPW - 1)
        plsc.store_scatter(tok_v, [drel_c], t, mask=m)
        return carry

    lax.fori_loop(0, (S * K) // 16, body, 0)

    for ch in range(RPW // GCH):
        idx = tok_v.at[pl.ds(ch * GCH, GCH)]
        pltpu.async_copy(x_hbm.at[idx], buf_v, sem).wait()
        pltpu.sync_copy(buf_v, xg_hbm.at[pl.ds(base + ch * GCH, GCH)])


def _ffn_kernel(bexp_ref, nblk_ref, xg_ref, w1_ref, b1_ref, w2_ref, b2_ref,
                out_ref):
    b = pl.program_id(0)

    @pl.when(b < nblk_ref[0])
    def _():
        h = jnp.dot(xg_ref[...], w1_ref[0], preferred_element_type=jnp.float32)
        h = h + b1_ref[0]
        h = 0.5 * h * (1.0 + lax.erf(h * (2.0 ** -0.5)))
        y = jnp.dot(h, w2_ref[0], preferred_element_type=jnp.float32)
        out_ref[...] = y + b2_ref[0]


def _combine_body(yg_hbm, dflat_hbm, no_hbm, ia_v, ib_v, ba_v, bb_v, bo_v,
                  sema, semb):
    wid = lax.axis_index("s") * NC + lax.axis_index("c")
    tbase = wid * TPW
    for ch in range(TPW // CCH):
        o = tbase + ch * CCH
        pltpu.sync_copy(dflat_hbm.at[pl.ds(o, CCH)], ia_v)
        pltpu.sync_copy(dflat_hbm.at[pl.ds(S + o, CCH)], ib_v)
        ca = pltpu.async_copy(yg_hbm.at[ia_v], ba_v, sema)
        cb = pltpu.async_copy(yg_hbm.at[ib_v], bb_v, semb)
        ca.wait()
        cb.wait()

        def row(r, carry):
            for cc in range(D // 16):
                sl = pl.ds(cc * 16, 16)
                bo_v[r, sl] = (ba_v[r, sl] + bb_v[r, sl]) * 0.5
            return carry

        lax.fori_loop(0, CCH, row, 0)
        pltpu.sync_copy(bo_v, no_hbm.at[pl.ds(o, CCH)])


def _qkv_kernel(no_ref, w_ref, b_ref, out_ref):
    out_ref[...] = lax.dot_general(
        no_ref[...], w_ref[...], (((1,), (1,)), ((), ())),
        preferred_element_type=jnp.float32) + b_ref[...]


def _attn_kernel(q_ref, k_ref, v_ref, out_ref):
    for i in range(2):
        cols = slice(i * DH, (i + 1) * DH)
        q = q_ref[:, cols]
        k = k_ref[:, cols]
        v = v_ref[:, cols]
        s = lax.dot_general(q, k, (((1,), (1,)), ((), ())),
                            preferred_element_type=jnp.float32)
        s = s * (1.0 / (DH ** 0.5))
        m = jnp.max(s, axis=1, keepdims=True)
        p = jnp.exp(s - m)
        p = p / jnp.sum(p, axis=1, keepdims=True)
        out_ref[:, cols] = jnp.dot(p, v, preferred_element_type=jnp.float32)


def _layernorm(x, w, b, eps=1e-5):
    mu = jnp.mean(x, axis=-1, keepdims=True)
    xc = x - mu
    var = jnp.mean(xc * xc, axis=-1, keepdims=True)
    return xc * jax.lax.rsqrt(var + eps) * w + b


def _final_kernel(x_ref, no_ref, ao_ref, w_ref, b_ref, l1w_ref, l1b_ref,
                  l2w_ref, l2b_ref, out_ref):
    attn_out = lax.dot_general(
        ao_ref[...], w_ref[...], (((1,), (1,)), ((), ())),
        preferred_element_type=jnp.float32) + b_ref[...]
    x1 = _layernorm(x_ref[...] + attn_out, l1w_ref[...], l1b_ref[...])
    out_ref[...] = _layernorm(x1 + no_ref[...], l2w_ref[...], l2b_ref[...])


def _gather_call():
    mesh = plsc.VectorSubcoreMesh(core_axis_name="c", subcore_axis_name="s",
                                  num_cores=NC, num_subcores=NS)
    return pl.kernel(
        _gather_body, mesh=mesh,
        compiler_params=pltpu.CompilerParams(needs_layout_passes=False),
        out_type=jax.ShapeDtypeStruct((P, D), jnp.float32),
        scratch_types=[
            pltpu.VMEM((S * K,), jnp.int32),
            pltpu.VMEM((RPW,), jnp.int32),
            pltpu.VMEM((GCH, D), jnp.float32),
            pltpu.SemaphoreType.DMA,
        ],
    )


def _combine_call():
    mesh = plsc.VectorSubcoreMesh(core_axis_name="c", subcore_axis_name="s",
                                  num_cores=NC, num_subcores=NS)
    return pl.kernel(
        _combine_body, mesh=mesh,
        compiler_params=pltpu.CompilerParams(needs_layout_passes=False),
        out_type=jax.ShapeDtypeStruct((S, D), jnp.float32),
        scratch_types=[
            pltpu.VMEM((CCH,), jnp.int32),
            pltpu.VMEM((CCH,), jnp.int32),
            pltpu.VMEM((CCH, D), jnp.float32),
            pltpu.VMEM((CCH, D), jnp.float32),
            pltpu.VMEM((CCH, D), jnp.float32),
            pltpu.SemaphoreType.DMA,
            pltpu.SemaphoreType.DMA,
        ],
    )


def kernel(x, router_w, router_b, W1, b1, W2, b2, in_proj_w, in_proj_b,
           out_proj_w, out_proj_b, ln1_w, ln1_b, ln2_w, ln2_b):
    x2 = x.reshape(S, D)
    rb2 = router_b.reshape(1, E)

    dest, bexp, nblk = pl.pallas_call(
        _dispatch_kernel,
        out_shape=(
            jax.ShapeDtypeStruct((S, K), jnp.int32),
            jax.ShapeDtypeStruct((NBLK, 1), jnp.int32),
            jax.ShapeDtypeStruct((1, 1), jnp.int32),
        ),
    )(x2, router_w, rb2)

    dflat = dest.T.reshape(S * K)

    xg = _gather_call()(x2, dflat)

    yg = pl.pallas_call(
        _ffn_kernel,
        grid_spec=pltpu.PrefetchScalarGridSpec(
            num_scalar_prefetch=2,
            grid=(NBLK,),
            in_specs=[
                pl.BlockSpec((BLK, D), lambda b, be, nb: (b, 0)),
                pl.BlockSpec((1, D, F), lambda b, be, nb: (be[b], 0, 0)),
                pl.BlockSpec((1, 1, F), lambda b, be, nb: (be[b], 0, 0)),
                pl.BlockSpec((1, F, D), lambda b, be, nb: (be[b], 0, 0)),
                pl.BlockSpec((1, 1, D), lambda b, be, nb: (be[b], 0, 0)),
            ],
            out_specs=pl.BlockSpec((BLK, D), lambda b, be, nb: (b, 0)),
        ),
        out_shape=jax.ShapeDtypeStruct((P, D), jnp.float32),
    )(bexp.reshape(NBLK), nblk.reshape(1), xg, W1, b1.reshape(E, 1, F), W2,
      b2.reshape(E, 1, D))

    neuron_outputs = _combine_call()(yg, dflat)

    qkv = pl.pallas_call(
        _qkv_kernel,
        out_shape=jax.ShapeDtypeStruct((S, 3 * D), jnp.float32),
    )(neuron_outputs, in_proj_w, in_proj_b.reshape(1, 3 * D))

    ao = pl.pallas_call(
        _attn_kernel,
        grid=(H // 2,),
        in_specs=[
            pl.BlockSpec((S, 2 * DH), lambda h: (0, h)),
            pl.BlockSpec((S, 2 * DH), lambda h: (0, H // 2 + h)),
            pl.BlockSpec((S, 2 * DH), lambda h: (0, H + h)),
        ],
        out_specs=pl.BlockSpec((S, 2 * DH), lambda h: (0, h)),
        out_shape=jax.ShapeDtypeStruct((S, D), jnp.float32),
    )(qkv, qkv, qkv)

    out = pl.pallas_call(
        _final_kernel,
        out_shape=jax.ShapeDtypeStruct((S, D), jnp.float32),
    )(x2, neuron_outputs, ao, out_proj_w, out_proj_b.reshape(1, D),
      ln1_w.reshape(1, D), ln1_b.reshape(1, D),
      ln2_w.reshape(1, D), ln2_b.reshape(1, D))

    return out.reshape(B, S, D)


# speedup vs baseline: 1.1101x; 1.0913x over previous
import functools

import jax
import jax.numpy as jnp
from jax import lax
from jax.experimental import pallas as pl
from jax.experimental.pallas import tpu as pltpu
from jax.experimental.pallas import tpu_sc as plsc

B, S, D, E, K, H = 1, 2048, 768, 8, 2, 12
F = 4 * D
DH = D // H

BLK = 128
NBLK = (S * K + E * (BLK - 1) + BLK - 1) // BLK
P = NBLK * BLK

NC, NS = 2, 16
NW = NC * NS
RPW = P // NW
GCH = RPW // 2
TPW = S // NW
CCH = 32

CB = 512


def _dispatch_kernel(x_ref, rw_ref, rb_ref, dest_ref, bexp_ref, nblk_ref):
    xb = x_ref[...]
    logits = jnp.dot(xb, rw_ref[...], preferred_element_type=jnp.float32)
    logits = logits + rb_ref[...]
    ecol = lax.broadcasted_iota(jnp.int32, (S, E), 1)
    cnt = jnp.zeros((S, E), jnp.float32)
    for j in range(E):
        lj = logits[:, j:j + 1]
        beats = (lj > logits) | ((lj == logits) & (j < ecol))
        cnt = cnt + beats.astype(jnp.float32)
    mask = jnp.where(cnt < float(K), 1.0, 0.0)

    r_i = lax.broadcasted_iota(jnp.int32, (CB, CB), 0)
    c_i = lax.broadcasted_iota(jnp.int32, (CB, CB), 1)
    tri = jnp.where(r_i >= c_i, 1.0, 0.0)
    carry = jnp.zeros((1, E), jnp.float32)
    chunks = []
    for c in range(S // CB):
        mc = mask[c * CB:(c + 1) * CB, :]
        posc = jnp.dot(tri, mc, preferred_element_type=jnp.float32) + carry
        carry = posc[CB - 1:CB, :]
        chunks.append(posc)
    pos = jnp.concatenate(chunks, axis=0)
    tot = pos[S - 1:S, :]

    pc = jnp.floor((tot + float(BLK - 1)) * (1.0 / BLK)) * float(BLK)
    r8 = lax.broadcasted_iota(jnp.int32, (E, E), 0)
    c8 = lax.broadcasted_iota(jnp.int32, (E, E), 1)
    ut = jnp.where(r8 < c8, 1.0, 0.0)
    off = jnp.dot(pc, ut, preferred_element_type=jnp.float32)

    slot = off + pos - 1.0
    d0 = jnp.sum(jnp.where(cnt == 0.0, slot, 0.0), axis=1, keepdims=True)
    d1 = jnp.sum(jnp.where(cnt == 1.0, slot, 0.0), axis=1, keepdims=True)
    dest_ref[...] = jnp.concatenate([d0, d1], axis=1).astype(jnp.int32)

    bB = lax.broadcasted_iota(jnp.int32, (NBLK, E), 0).astype(jnp.float32)
    bB = bB * float(BLK)
    ind = jnp.where((bB >= off) & (bB < off + pc), 1.0, 0.0)
    ecolf = lax.broadcasted_iota(jnp.int32, (NBLK, E), 1).astype(jnp.float32)
    val = jnp.sum(ind * ecolf, axis=1, keepdims=True)
    has = jnp.sum(ind, axis=1, keepdims=True)
    bexp_ref[...] = (val + float(E - 1) * (1.0 - has)).astype(jnp.int32)
    nblk_ref[...] = (jnp.sum(pc, axis=1, keepdims=True) *
                     (1.0 / BLK)).astype(jnp.int32)


def _gather_body(x_hbm, dflat_hbm, xg_hbm, dflat_v, tok_v, bufa_v, bufb_v,
                 gsa, gsb, wsa, wsb):
    wid = lax.axis_index("s") * NC + lax.axis_index("c")
    base = wid * RPW
    pltpu.sync_copy(dflat_hbm, dflat_v)
    for cc in range(RPW // 16):
        tok_v[pl.ds(cc * 16, 16)] = jnp.zeros((16,), jnp.int32)

    def body(i, carry):
        d = dflat_v[pl.ds(i * 16, 16)]
        a = lax.iota(jnp.int32, 16) + i * 16
        t = lax.bitwise_and(a, S - 1)
        drel = d - base
        m = (drel >= 0) & (drel < RPW)
        drel_c = jnp.clip(drel, 0, RPW - 1)
        plsc.store_scatter(tok_v, [drel_c], t, mask=m)
        return carry

    lax.fori_loop(0, (S * K) // 16, body, 0, unroll=8)

    g0 = pltpu.async_copy(x_hbm.at[tok_v.at[pl.ds(0, GCH)]], bufa_v, gsa)
    g1 = pltpu.async_copy(x_hbm.at[tok_v.at[pl.ds(GCH, GCH)]], bufb_v, gsb)
    g0.wait()
    w0 = pltpu.async_copy(bufa_v, xg_hbm.at[pl.ds(base, GCH)], wsa)
    g1.wait()
    w1 = pltpu.async_copy(bufb_v, xg_hbm.at[pl.ds(base + GCH, GCH)], wsb)
    w0.wait()
    w1.wait()


def _ffn_kernel(bexp_ref, nblk_ref, xg_ref, w1_ref, b1_ref, w2_ref, b2_ref,
                out_ref):
    b = pl.program_id(0)

    @pl.when(b < nblk_ref[0])
    def _():
        h = jnp.dot(xg_ref[...], w1_ref[0], preferred_element_type=jnp.float32)
        h = h + b1_ref[0]
        h = 0.5 * h * (1.0 + lax.erf(h * (2.0 ** -0.5)))
        y = jnp.dot(h, w2_ref[0], preferred_element_type=jnp.float32)
        out_ref[...] = y + b2_ref[0]


def _combine_body(yg_hbm, dflat_hbm, no_hbm, ia_v, ib_v, ba_v, bb_v, bo_v,
                  sema, semb):
    wid = lax.axis_index("s") * NC + lax.axis_index("c")
    tbase = wid * TPW
    for ch in range(TPW // CCH):
        o = tbase + ch * CCH
        pltpu.sync_copy(dflat_hbm.at[pl.ds(o, CCH)], ia_v)
        pltpu.sync_copy(dflat_hbm.at[pl.ds(S + o, CCH)], ib_v)
        ca = pltpu.async_copy(yg_hbm.at[ia_v], ba_v, sema)
        cb = pltpu.async_copy(yg_hbm.at[ib_v], bb_v, semb)
        ca.wait()
        cb.wait()

        def row(r, carry):
            for cc in range(D // 16):
                sl = pl.ds(cc * 16, 16)
                bo_v[r, sl] = (ba_v[r, sl] + bb_v[r, sl]) * 0.5
            return carry

        lax.fori_loop(0, CCH, row, 0)
        pltpu.sync_copy(bo_v, no_hbm.at[pl.ds(o, CCH)])


def _qkv_kernel(no_ref, w_ref, b_ref, out_ref):
    out_ref[...] = lax.dot_general(
        no_ref[...], w_ref[...], (((1,), (1,)), ((), ())),
        preferred_element_type=jnp.float32) + b_ref[...]


def _attn_kernel(q_ref, k_ref, v_ref, out_ref):
    for i in range(2):
        cols = slice(i * DH, (i + 1) * DH)
        q = q_ref[:, cols]
        k = k_ref[:, cols]
        v = v_ref[:, cols]
        s = lax.dot_general(q, k, (((1,), (1,)), ((), ())),
                            preferred_element_type=jnp.float32)
        s = s * (1.0 / (DH ** 0.5))
        m = jnp.max(s, axis=1, keepdims=True)
        p = jnp.exp(s - m)
        p = p / jnp.sum(p, axis=1, keepdims=True)
        out_ref[:, cols] = jnp.dot(p, v, preferred_element_type=jnp.float32)


def _layernorm(x, w, b, eps=1e-5):
    mu = jnp.mean(x, axis=-1, keepdims=True)
    xc = x - mu
    var = jnp.mean(xc * xc, axis=-1, keepdims=True)
    return xc * jax.lax.rsqrt(var + eps) * w + b


def _final_kernel(x_ref, no_ref, ao_ref, w_ref, b_ref, l1w_ref, l1b_ref,
                  l2w_ref, l2b_ref, out_ref):
    attn_out = lax.dot_general(
        ao_ref[...], w_ref[...], (((1,), (1,)), ((), ())),
        preferred_element_type=jnp.float32) + b_ref[...]
    x1 = _layernorm(x_ref[...] + attn_out, l1w_ref[...], l1b_ref[...])
    out_ref[...] = _layernorm(x1 + no_ref[...], l2w_ref[...], l2b_ref[...])


def _gather_call():
    mesh = plsc.VectorSubcoreMesh(core_axis_name="c", subcore_axis_name="s",
                                  num_cores=NC, num_subcores=NS)
    return pl.kernel(
        _gather_body, mesh=mesh,
        compiler_params=pltpu.CompilerParams(needs_layout_passes=False),
        out_type=jax.ShapeDtypeStruct((P, D), jnp.float32),
        scratch_types=[
            pltpu.VMEM((S * K,), jnp.int32),
            pltpu.VMEM((RPW,), jnp.int32),
            pltpu.VMEM((GCH, D), jnp.float32),
            pltpu.VMEM((GCH, D), jnp.float32),
            pltpu.SemaphoreType.DMA,
            pltpu.SemaphoreType.DMA,
            pltpu.SemaphoreType.DMA,
            pltpu.SemaphoreType.DMA,
        ],
    )


def _combine_call():
    mesh = plsc.VectorSubcoreMesh(core_axis_name="c", subcore_axis_name="s",
                                  num_cores=NC, num_subcores=NS)
    return pl.kernel(
        _combine_body, mesh=mesh,
        compiler_params=pltpu.CompilerParams(needs_layout_passes=False),
        out_type=jax.ShapeDtypeStruct((S, D), jnp.float32),
        scratch_types=[
            pltpu.VMEM((CCH,), jnp.int32),
            pltpu.VMEM((CCH,), jnp.int32),
            pltpu.VMEM((CCH, D), jnp.float32),
            pltpu.VMEM((CCH, D), jnp.float32),
            pltpu.VMEM((CCH, D), jnp.float32),
            pltpu.SemaphoreType.DMA,
            pltpu.SemaphoreType.DMA,
        ],
    )


def kernel(x, router_w, router_b, W1, b1, W2, b2, in_proj_w, in_proj_b,
           out_proj_w, out_proj_b, ln1_w, ln1_b, ln2_w, ln2_b):
    x2 = x.reshape(S, D)
    rb2 = router_b.reshape(1, E)

    dest, bexp, nblk = pl.pallas_call(
        _dispatch_kernel,
        out_shape=(
            jax.ShapeDtypeStruct((S, K), jnp.int32),
            jax.ShapeDtypeStruct((NBLK, 1), jnp.int32),
            jax.ShapeDtypeStruct((1, 1), jnp.int32),
        ),
    )(x2, router_w, rb2)

    dflat = dest.T.reshape(S * K)

    xg = _gather_call()(x2, dflat)

    yg = pl.pallas_call(
        _ffn_kernel,
        grid_spec=pltpu.PrefetchScalarGridSpec(
            num_scalar_prefetch=2,
            grid=(NBLK,),
            in_specs=[
                pl.BlockSpec((BLK, D), lambda b, be, nb: (b, 0)),
                pl.BlockSpec((1, D, F), lambda b, be, nb: (be[b], 0, 0)),
                pl.BlockSpec((1, 1, F), lambda b, be, nb: (be[b], 0, 0)),
                pl.BlockSpec((1, F, D), lambda b, be, nb: (be[b], 0, 0)),
                pl.BlockSpec((1, 1, D), lambda b, be, nb: (be[b], 0, 0)),
            ],
            out_specs=pl.BlockSpec((BLK, D), lambda b, be, nb: (b, 0)),
        ),
        out_shape=jax.ShapeDtypeStruct((P, D), jnp.float32),
    )(bexp.reshape(NBLK), nblk.reshape(1), xg, W1, b1.reshape(E, 1, F), W2,
      b2.reshape(E, 1, D))

    neuron_outputs = _combine_call()(yg, dflat)

    qkv = pl.pallas_call(
        _qkv_kernel,
        out_shape=jax.ShapeDtypeStruct((S, 3 * D), jnp.float32),
    )(neuron_outputs, in_proj_w, in_proj_b.reshape(1, 3 * D))

    ao = pl.pallas_call(
        _attn_kernel,
        grid=(H // 2,),
        in_specs=[
            pl.BlockSpec((S, 2 * DH), lambda h: (0, h)),
            pl.BlockSpec((S, 2 * DH), lambda h: (0, H // 2 + h)),
            pl.BlockSpec((S, 2 * DH), lambda h: (0, H + h)),
        ],
        out_specs=pl.BlockSpec((S, 2 * DH), lambda h: (0, h)),
        out_shape=jax.ShapeDtypeStruct((S, D), jnp.float32),
    )(qkv, qkv, qkv)

    out = pl.pallas_call(
        _final_kernel,
        out_shape=jax.ShapeDtypeStruct((S, D), jnp.float32),
    )(x2, neuron_outputs, ao, out_proj_w, out_proj_b.reshape(1, D),
      ln1_w.reshape(1, D), ln1_b.reshape(1, D),
      ln2_w.reshape(1, D), ln2_b.reshape(1, D))

    return out.reshape(B, S, D)


# speedup vs baseline: 1.3162x; 1.1857x over previous
import functools

import jax
import jax.numpy as jnp
from jax import lax
from jax.experimental import pallas as pl
from jax.experimental.pallas import tpu as pltpu
from jax.experimental.pallas import tpu_sc as plsc

B, S, D, E, K, H = 1, 2048, 768, 8, 2, 12
F = 4 * D
DH = D // H

BLK = 128
NBLK = (S * K + E * (BLK - 1) + BLK - 1) // BLK
P = NBLK * BLK

NC, NS = 2, 16
NW = NC * NS
APW = (S * K) // NW
TPW = S // NW
CCH = 32

CB = 512


def _dispatch_kernel(x_ref, rw_ref, rb_ref, dest_ref, bexp_ref, nblk_ref):
    xb = x_ref[...]
    logits = jnp.dot(xb, rw_ref[...], preferred_element_type=jnp.float32)
    logits = logits + rb_ref[...]
    ecol = lax.broadcasted_iota(jnp.int32, (S, E), 1)
    cnt = jnp.zeros((S, E), jnp.float32)
    for j in range(E):
        lj = logits[:, j:j + 1]
        beats = (lj > logits) | ((lj == logits) & (j < ecol))
        cnt = cnt + beats.astype(jnp.float32)
    mask = jnp.where(cnt < float(K), 1.0, 0.0)

    r_i = lax.broadcasted_iota(jnp.int32, (CB, CB), 0)
    c_i = lax.broadcasted_iota(jnp.int32, (CB, CB), 1)
    tri = jnp.where(r_i >= c_i, 1.0, 0.0)
    carry = jnp.zeros((1, E), jnp.float32)
    chunks = []
    for c in range(S // CB):
        mc = mask[c * CB:(c + 1) * CB, :]
        posc = jnp.dot(tri, mc, preferred_element_type=jnp.float32) + carry
        carry = posc[CB - 1:CB, :]
        chunks.append(posc)
    pos = jnp.concatenate(chunks, axis=0)
    tot = pos[S - 1:S, :]

    pc = jnp.floor((tot + float(BLK - 1)) * (1.0 / BLK)) * float(BLK)
    r8 = lax.broadcasted_iota(jnp.int32, (E, E), 0)
    c8 = lax.broadcasted_iota(jnp.int32, (E, E), 1)
    ut = jnp.where(r8 < c8, 1.0, 0.0)
    off = jnp.dot(pc, ut, preferred_element_type=jnp.float32)

    slot = off + pos - 1.0
    d0 = jnp.sum(jnp.where(cnt == 0.0, slot, 0.0), axis=1, keepdims=True)
    d1 = jnp.sum(jnp.where(cnt == 1.0, slot, 0.0), axis=1, keepdims=True)
    dest_ref[...] = jnp.concatenate([d0, d1], axis=1).astype(jnp.int32)

    bB = lax.broadcasted_iota(jnp.int32, (NBLK, E), 0).astype(jnp.float32)
    bB = bB * float(BLK)
    ind = jnp.where((bB >= off) & (bB < off + pc), 1.0, 0.0)
    ecolf = lax.broadcasted_iota(jnp.int32, (NBLK, E), 1).astype(jnp.float32)
    val = jnp.sum(ind * ecolf, axis=1, keepdims=True)
    has = jnp.sum(ind, axis=1, keepdims=True)
    bexp_ref[...] = (val + float(E - 1) * (1.0 - has)).astype(jnp.int32)
    nblk_ref[...] = (jnp.sum(pc, axis=1, keepdims=True) *
                     (1.0 / BLK)).astype(jnp.int32)


def _gather_body(x_hbm, d2_hbm, xg_hbm, idx_v, buf_v, isem, xsem, wsem):
    wid = lax.axis_index("s") * NC + lax.axis_index("c")
    t0 = lax.bitwise_and(wid, NW // K - 1) * APW
    ci = pltpu.async_copy(d2_hbm.at[wid], idx_v, isem)
    cx = pltpu.async_copy(x_hbm.at[pl.ds(t0, APW)], buf_v, xsem)
    ci.wait()
    cx.wait()
    pltpu.async_copy(buf_v, xg_hbm.at[idx_v], wsem).wait()


def _ffn_kernel(bexp_ref, nblk_ref, xg_ref, w1_ref, b1_ref, w2_ref, b2_ref,
                out_ref):
    b = pl.program_id(0)

    @pl.when(b < nblk_ref[0])
    def _():
        h = jnp.dot(xg_ref[...], w1_ref[0], preferred_element_type=jnp.float32)
        h = h + b1_ref[0]
        h = 0.5 * h * (1.0 + lax.erf(h * (2.0 ** -0.5)))
        y = jnp.dot(h, w2_ref[0], preferred_element_type=jnp.float32)
        out_ref[...] = y + b2_ref[0]


def _combine_body(yg_hbm, dflat_hbm, no_hbm, ia_v, ib_v, ba_v, bb_v, bo_v,
                  sema, semb):
    wid = lax.axis_index("s") * NC + lax.axis_index("c")
    tbase = wid * TPW
    for ch in range(TPW // CCH):
        o = tbase + ch * CCH
        pltpu.sync_copy(dflat_hbm.at[pl.ds(o, CCH)], ia_v)
        pltpu.sync_copy(dflat_hbm.at[pl.ds(S + o, CCH)], ib_v)
        ca = pltpu.async_copy(yg_hbm.at[ia_v], ba_v, sema)
        cb = pltpu.async_copy(yg_hbm.at[ib_v], bb_v, semb)
        ca.wait()
        cb.wait()

        def row(r, carry):
            for cc in range(D // 16):
                sl = pl.ds(cc * 16, 16)
                bo_v[r, sl] = (ba_v[r, sl] + bb_v[r, sl]) * 0.5
            return carry

        lax.fori_loop(0, CCH, row, 0)
        pltpu.sync_copy(bo_v, no_hbm.at[pl.ds(o, CCH)])


def _qkv_kernel(no_ref, w_ref, b_ref, out_ref):
    out_ref[...] = lax.dot_general(
        no_ref[...], w_ref[...], (((1,), (1,)), ((), ())),
        preferred_element_type=jnp.float32) + b_ref[...]


def _attn_kernel(q_ref, k_ref, v_ref, out_ref):
    for i in range(2):
        cols = slice(i * DH, (i + 1) * DH)
        q = q_ref[:, cols]
        k = k_ref[:, cols]
        v = v_ref[:, cols]
        s = lax.dot_general(q, k, (((1,), (1,)), ((), ())),
                            preferred_element_type=jnp.float32)
        s = s * (1.0 / (DH ** 0.5))
        m = jnp.max(s, axis=1, keepdims=True)
        p = jnp.exp(s - m)
        p = p / jnp.sum(p, axis=1, keepdims=True)
        out_ref[:, cols] = jnp.dot(p, v, preferred_element_type=jnp.float32)


def _layernorm(x, w, b, eps=1e-5):
    mu = jnp.mean(x, axis=-1, keepdims=True)
    xc = x - mu
    var = jnp.mean(xc * xc, axis=-1, keepdims=True)
    return xc * jax.lax.rsqrt(var + eps) * w + b


def _final_kernel(x_ref, no_ref, ao_ref, w_ref, b_ref, l1w_ref, l1b_ref,
                  l2w_ref, l2b_ref, out_ref):
    attn_out = lax.dot_general(
        ao_ref[...], w_ref[...], (((1,), (1,)), ((), ())),
        preferred_element_type=jnp.float32) + b_ref[...]
    x1 = _layernorm(x_ref[...] + attn_out, l1w_ref[...], l1b_ref[...])
    out_ref[...] = _layernorm(x1 + no_ref[...], l2w_ref[...], l2b_ref[...])


def _gather_call():
    mesh = plsc.VectorSubcoreMesh(core_axis_name="c", subcore_axis_name="s",
                                  num_cores=NC, num_subcores=NS)
    return pl.kernel(
        _gather_body, mesh=mesh,
        compiler_params=pltpu.CompilerParams(needs_layout_passes=False),
        out_type=jax.ShapeDtypeStruct((P, D), jnp.float32),
        scratch_types=[
            pltpu.VMEM((APW,), jnp.int32),
            pltpu.VMEM((APW, D), jnp.float32),
            pltpu.SemaphoreType.DMA,
            pltpu.SemaphoreType.DMA,
            pltpu.SemaphoreType.DMA,
        ],
    )


def _combine_call():
    mesh = plsc.VectorSubcoreMesh(core_axis_name="c", subcore_axis_name="s",
                                  num_cores=NC, num_subcores=NS)
    return pl.kernel(
        _combine_body, mesh=mesh,
        compiler_params=pltpu.CompilerParams(needs_layout_passes=False),
        out_type=jax.ShapeDtypeStruct((S, D), jnp.float32),
        scratch_types=[
            pltpu.VMEM((CCH,), jnp.int32),
            pltpu.VMEM((CCH,), jnp.int32),
            pltpu.VMEM((CCH, D), jnp.float32),
            pltpu.VMEM((CCH, D), jnp.float32),
            pltpu.VMEM((CCH, D), jnp.float32),
            pltpu.SemaphoreType.DMA,
            pltpu.SemaphoreType.DMA,
        ],
    )


def kernel(x, router_w, router_b, W1, b1, W2, b2, in_proj_w, in_proj_b,
           out_proj_w, out_proj_b, ln1_w, ln1_b, ln2_w, ln2_b):
    x2 = x.reshape(S, D)
    rb2 = router_b.reshape(1, E)

    dest, bexp, nblk = pl.pallas_call(
        _dispatch_kernel,
        out_shape=(
            jax.ShapeDtypeStruct((S, K), jnp.int32),
            jax.ShapeDtypeStruct((NBLK, 1), jnp.int32),
            jax.ShapeDtypeStruct((1, 1), jnp.int32),
        ),
    )(x2, router_w, rb2)

    dflat = dest.T.reshape(S * K)

    xg = _gather_call()(x2, dflat.reshape(NW, APW))

    yg = pl.pallas_call(
        _ffn_kernel,
        grid_spec=pltpu.PrefetchScalarGridSpec(
            num_scalar_prefetch=2,
            grid=(NBLK,),
            in_specs=[
                pl.BlockSpec((BLK, D), lambda b, be, nb: (b, 0)),
                pl.BlockSpec((1, D, F), lambda b, be, nb: (be[b], 0, 0)),
                pl.BlockSpec((1, 1, F), lambda b, be, nb: (be[b], 0, 0)),
                pl.BlockSpec((1, F, D), lambda b, be, nb: (be[b], 0, 0)),
                pl.BlockSpec((1, 1, D), lambda b, be, nb: (be[b], 0, 0)),
            ],
            out_specs=pl.BlockSpec((BLK, D), lambda b, be, nb: (b, 0)),
        ),
        out_shape=jax.ShapeDtypeStruct((P, D), jnp.float32),
    )(bexp.reshape(NBLK), nblk.reshape(1), xg, W1, b1.reshape(E, 1, F), W2,
      b2.reshape(E, 1, D))

    neuron_outputs = _combine_call()(yg, dflat)

    qkv = pl.pallas_call(
        _qkv_kernel,
        out_shape=jax.ShapeDtypeStruct((S, 3 * D), jnp.float32),
    )(neuron_outputs, in_proj_w, in_proj_b.reshape(1, 3 * D))

    ao = pl.pallas_call(
        _attn_kernel,
        grid=(H // 2,),
        in_specs=[
            pl.BlockSpec((S, 2 * DH), lambda h: (0, h)),
            pl.BlockSpec((S, 2 * DH), lambda h: (0, H // 2 + h)),
            pl.BlockSpec((S, 2 * DH), lambda h: (0, H + h)),
        ],
        out_specs=pl.BlockSpec((S, 2 * DH), lambda h: (0, h)),
        out_shape=jax.ShapeDtypeStruct((S, D), jnp.float32),
    )(qkv, qkv, qkv)

    out = pl.pallas_call(
        _final_kernel,
        out_shape=jax.ShapeDtypeStruct((S, D), jnp.float32),
    )(x2, neuron_outputs, ao, out_proj_w, out_proj_b.reshape(1, D),
      ln1_w.reshape(1, D), ln1_b.reshape(1, D),
      ln2_w.reshape(1, D), ln2_b.reshape(1, D))

    return out.reshape(B, S, D)


# speedup vs baseline: 1.4050x; 1.0674x over previous
import functools

import jax
import jax.numpy as jnp
from jax import lax
from jax.experimental import pallas as pl
from jax.experimental.pallas import tpu as pltpu
from jax.experimental.pallas import tpu_sc as plsc

B, S, D, E, K, H = 1, 2048, 768, 8, 2, 12
F = 4 * D
DH = D // H

BLK = 128
NBLK = (S * K + E * (BLK - 1) + BLK - 1) // BLK
P = NBLK * BLK

NC, NS = 2, 16
NW = NC * NS
APW = (S * K) // NW
TPW = S // NW
CCH = 32

CB = 512


def _dispatch_kernel(x_ref, rw_ref, rb_ref, dest_ref, bexp_ref, nblk_ref):
    xb = x_ref[...]
    logits = jnp.dot(xb, rw_ref[...], preferred_element_type=jnp.float32)
    logits = logits + rb_ref[...]
    ecol = lax.broadcasted_iota(jnp.int32, (S, E), 1)
    cnt = jnp.zeros((S, E), jnp.float32)
    for j in range(E):
        lj = logits[:, j:j + 1]
        beats = (lj > logits) | ((lj == logits) & (j < ecol))
        cnt = cnt + beats.astype(jnp.float32)
    mask = jnp.where(cnt < float(K), 1.0, 0.0)

    r_i = lax.broadcasted_iota(jnp.int32, (CB, CB), 0)
    c_i = lax.broadcasted_iota(jnp.int32, (CB, CB), 1)
    tri = jnp.where(r_i >= c_i, 1.0, 0.0)
    carry = jnp.zeros((1, E), jnp.float32)
    chunks = []
    for c in range(S // CB):
        mc = mask[c * CB:(c + 1) * CB, :]
        posc = jnp.dot(tri, mc, preferred_element_type=jnp.float32) + carry
        carry = posc[CB - 1:CB, :]
        chunks.append(posc)
    pos = jnp.concatenate(chunks, axis=0)
    tot = pos[S - 1:S, :]

    pc = jnp.floor((tot + float(BLK - 1)) * (1.0 / BLK)) * float(BLK)
    r8 = lax.broadcasted_iota(jnp.int32, (E, E), 0)
    c8 = lax.broadcasted_iota(jnp.int32, (E, E), 1)
    ut = jnp.where(r8 < c8, 1.0, 0.0)
    off = jnp.dot(pc, ut, preferred_element_type=jnp.float32)

    slot = off + pos - 1.0
    d0 = jnp.sum(jnp.where(cnt == 0.0, slot, 0.0), axis=1, keepdims=True)
    d1 = jnp.sum(jnp.where(cnt == 1.0, slot, 0.0), axis=1, keepdims=True)
    dest_ref[...] = jnp.concatenate([d0, d1], axis=1).astype(jnp.int32)

    bB = lax.broadcasted_iota(jnp.int32, (NBLK, E), 0).astype(jnp.float32)
    bB = bB * float(BLK)
    ind = jnp.where((bB >= off) & (bB < off + pc), 1.0, 0.0)
    ecolf = lax.broadcasted_iota(jnp.int32, (NBLK, E), 1).astype(jnp.float32)
    val = jnp.sum(ind * ecolf, axis=1, keepdims=True)
    has = jnp.sum(ind, axis=1, keepdims=True)
    bexp_ref[...] = (val + float(E - 1) * (1.0 - has)).astype(jnp.int32)
    nblk_ref[...] = (jnp.sum(pc, axis=1, keepdims=True) *
                     (1.0 / BLK)).astype(jnp.int32)


def _gather_body(x_hbm, d2_hbm, xg_hbm, idx_v, buf_v, isem, xsem, wsem):
    wid = lax.axis_index("s") * NC + lax.axis_index("c")
    t0 = lax.bitwise_and(wid, NW // K - 1) * APW
    ci = pltpu.async_copy(d2_hbm.at[wid], idx_v, isem)
    cx = pltpu.async_copy(x_hbm.at[pl.ds(t0, APW)], buf_v, xsem)
    ci.wait()
    cx.wait()
    pltpu.async_copy(buf_v, xg_hbm.at[idx_v], wsem).wait()


def _ffn_kernel(bexp_ref, nblk_ref, xg_ref, w1_ref, b1_ref, w2_ref, b2_ref,
                out_ref):
    b = pl.program_id(0)

    @pl.when(b < nblk_ref[0])
    def _():
        h = jnp.dot(xg_ref[...], w1_ref[0], preferred_element_type=jnp.float32)
        h = h + b1_ref[0]
        h = 0.5 * h * (1.0 + lax.erf(h * (2.0 ** -0.5)))
        y = jnp.dot(h, w2_ref[0], preferred_element_type=jnp.float32)
        out_ref[...] = y + b2_ref[0]


def _combine_body(yg_hbm, dflat_hbm, no_hbm, ia_v, ib_v, ba_v, bb_v, bo_v,
                  sema, semb):
    wid = lax.axis_index("s") * NC + lax.axis_index("c")
    tbase = wid * TPW
    for ch in range(TPW // CCH):
        o = tbase + ch * CCH
        pltpu.sync_copy(dflat_hbm.at[pl.ds(o, CCH)], ia_v)
        pltpu.sync_copy(dflat_hbm.at[pl.ds(S + o, CCH)], ib_v)
        ca = pltpu.async_copy(yg_hbm.at[ia_v], ba_v, sema)
        cb = pltpu.async_copy(yg_hbm.at[ib_v], bb_v, semb)
        ca.wait()
        cb.wait()

        def row(r, carry):
            for cc in range(D // 16):
                sl = pl.ds(cc * 16, 16)
                bo_v[r, sl] = (ba_v[r, sl] + bb_v[r, sl]) * 0.5
            return carry

        lax.fori_loop(0, CCH, row, 0, unroll=4)
        pltpu.sync_copy(bo_v, no_hbm.at[pl.ds(o, CCH)])


def _attn_kernel(no_ref, wq_ref, wk_ref, wv_ref, bq_ref, bk_ref, bv_ref,
                 out_ref):
    no = no_ref[...]
    cdims = (((1,), (1,)), ((), ()))
    qq = lax.dot_general(no, wq_ref[...], cdims,
                         preferred_element_type=jnp.float32) + bq_ref[...]
    kk = lax.dot_general(no, wk_ref[...], cdims,
                         preferred_element_type=jnp.float32) + bk_ref[...]
    vv = lax.dot_general(no, wv_ref[...], cdims,
                         preferred_element_type=jnp.float32) + bv_ref[...]
    for i in range(2):
        cols = slice(i * DH, (i + 1) * DH)
        q = qq[:, cols]
        k = kk[:, cols]
        v = vv[:, cols]
        s = lax.dot_general(q, k, cdims, preferred_element_type=jnp.float32)
        s = s * (1.0 / (DH ** 0.5))
        m = jnp.max(s, axis=1, keepdims=True)
        p = jnp.exp(s - m)
        r = 1.0 / jnp.sum(p, axis=1, keepdims=True)
        ao = jnp.dot(p, v, preferred_element_type=jnp.float32)
        out_ref[:, cols] = ao * r


def _layernorm(x, w, b, eps=1e-5):
    mu = jnp.mean(x, axis=-1, keepdims=True)
    xc = x - mu
    var = jnp.mean(xc * xc, axis=-1, keepdims=True)
    return xc * jax.lax.rsqrt(var + eps) * w + b


def _final_kernel(x_ref, no_ref, ao_ref, w_ref, b_ref, l1w_ref, l1b_ref,
                  l2w_ref, l2b_ref, out_ref):
    attn_out = lax.dot_general(
        ao_ref[...], w_ref[...], (((1,), (1,)), ((), ())),
        preferred_element_type=jnp.float32) + b_ref[...]
    x1 = _layernorm(x_ref[...] + attn_out, l1w_ref[...], l1b_ref[...])
    out_ref[...] = _layernorm(x1 + no_ref[...], l2w_ref[...], l2b_ref[...])


def _gather_call():
    mesh = plsc.VectorSubcoreMesh(core_axis_name="c", subcore_axis_name="s",
                                  num_cores=NC, num_subcores=NS)
    return pl.kernel(
        _gather_body, mesh=mesh,
        compiler_params=pltpu.CompilerParams(needs_layout_passes=False),
        out_type=jax.ShapeDtypeStruct((P, D), jnp.float32),
        scratch_types=[
            pltpu.VMEM((APW,), jnp.int32),
            pltpu.VMEM((APW, D), jnp.float32),
            pltpu.SemaphoreType.DMA,
            pltpu.SemaphoreType.DMA,
            pltpu.SemaphoreType.DMA,
        ],
    )


def _combine_call():
    mesh = plsc.VectorSubcoreMesh(core_axis_name="c", subcore_axis_name="s",
                                  num_cores=NC, num_subcores=NS)
    return pl.kernel(
        _combine_body, mesh=mesh,
        compiler_params=pltpu.CompilerParams(needs_layout_passes=False),
        out_type=jax.ShapeDtypeStruct((S, D), jnp.float32),
        scratch_types=[
            pltpu.VMEM((CCH,), jnp.int32),
            pltpu.VMEM((CCH,), jnp.int32),
            pltpu.VMEM((CCH, D), jnp.float32),
            pltpu.VMEM((CCH, D), jnp.float32),
            pltpu.VMEM((CCH, D), jnp.float32),
            pltpu.SemaphoreType.DMA,
            pltpu.SemaphoreType.DMA,
        ],
    )


def kernel(x, router_w, router_b, W1, b1, W2, b2, in_proj_w, in_proj_b,
           out_proj_w, out_proj_b, ln1_w, ln1_b, ln2_w, ln2_b):
    x2 = x.reshape(S, D)
    rb2 = router_b.reshape(1, E)

    dest, bexp, nblk = pl.pallas_call(
        _dispatch_kernel,
        out_shape=(
            jax.ShapeDtypeStruct((S, K), jnp.int32),
            jax.ShapeDtypeStruct((NBLK, 1), jnp.int32),
            jax.ShapeDtypeStruct((1, 1), jnp.int32),
        ),
    )(x2, router_w, rb2)

    dflat = dest.T.reshape(S * K)

    xg = _gather_call()(x2, dflat.reshape(NW, APW))

    yg = pl.pallas_call(
        _ffn_kernel,
        grid_spec=pltpu.PrefetchScalarGridSpec(
            num_scalar_prefetch=2,
            grid=(NBLK,),
            in_specs=[
                pl.BlockSpec((BLK, D), lambda b, be, nb: (b, 0)),
                pl.BlockSpec((1, D, F), lambda b, be, nb: (be[b], 0, 0)),
                pl.BlockSpec((1, 1, F), lambda b, be, nb: (be[b], 0, 0)),
                pl.BlockSpec((1, F, D), lambda b, be, nb: (be[b], 0, 0)),
                pl.BlockSpec((1, 1, D), lambda b, be, nb: (be[b], 0, 0)),
            ],
            out_specs=pl.BlockSpec((BLK, D), lambda b, be, nb: (b, 0)),
        ),
        out_shape=jax.ShapeDtypeStruct((P, D), jnp.float32),
    )(bexp.reshape(NBLK), nblk.reshape(1), xg, W1, b1.reshape(E, 1, F), W2,
      b2.reshape(E, 1, D))

    neuron_outputs = _combine_call()(yg, dflat)

    ipb = in_proj_b.reshape(1, 3 * D)
    nh2 = H // 2
    ao = pl.pallas_call(
        _attn_kernel,
        grid=(nh2,),
        in_specs=[
            pl.BlockSpec((S, D), lambda h: (0, 0)),
            pl.BlockSpec((2 * DH, D), lambda h: (h, 0)),
            pl.BlockSpec((2 * DH, D), lambda h: (nh2 + h, 0)),
            pl.BlockSpec((2 * DH, D), lambda h: (2 * nh2 + h, 0)),
            pl.BlockSpec((1, 2 * DH), lambda h: (0, h)),
            pl.BlockSpec((1, 2 * DH), lambda h: (0, nh2 + h)),
            pl.BlockSpec((1, 2 * DH), lambda h: (0, 2 * nh2 + h)),
        ],
        out_specs=pl.BlockSpec((S, 2 * DH), lambda h: (0, h)),
        out_shape=jax.ShapeDtypeStruct((S, D), jnp.float32),
    )(neuron_outputs, in_proj_w, in_proj_w, in_proj_w, ipb, ipb, ipb)

    out = pl.pallas_call(
        _final_kernel,
        out_shape=jax.ShapeDtypeStruct((S, D), jnp.float32),
    )(x2, neuron_outputs, ao, out_proj_w, out_proj_b.reshape(1, D),
      ln1_w.reshape(1, D), ln1_b.reshape(1, D),
      ln2_w.reshape(1, D), ln2_b.reshape(1, D))

    return out.reshape(B, S, D)
